# baseline (speedup 1.0000x reference)
import functools

import jax
import jax.numpy as jnp
from jax import lax
from jax.experimental import pallas as pl
from jax.experimental.pallas import tpu as pltpu
from jax.experimental.pallas import tpu_sc as plsc

_R = 20000
_C = 91
_CP = 128
_NP = _R * _CP
_NC = 2
_NS = 16
_NW = _NC * _NS
_L = 16
_U = 8
_CHUNK = _NP // _NW
_NEG = -1.0e30
_BR = 2000


def _tc_pad_body(x_ref, out_ref):
    pad = jnp.full((_BR, _CP - _C), _NEG, jnp.float32)
    out_ref[...] = jnp.concatenate([x_ref[0], pad], axis=1)


def _tc_pad(pred_logits):
    return pl.pallas_call(
        _tc_pad_body,
        grid=(_R // _BR,),
        in_specs=[pl.BlockSpec((1, _BR, _C), lambda i: (0, i, 0))],
        out_specs=pl.BlockSpec((_BR, _CP), lambda i: (i, 0)),
        out_shape=jax.ShapeDtypeStruct((_R, _CP), jnp.float32),
    )(pred_logits)


def _sc_scan_body(x_hbm, vals_out, idxs_out, buf, val_s, idx_s):
    cid = lax.axis_index("c")
    sid = lax.axis_index("s")
    wid = sid * _NC + cid
    base = wid * _CHUNK
    pltpu.sync_copy(x_hbm.at[pl.ds(base, _CHUNK)], buf)

    lanes = lax.iota(jnp.int32, 16)

    def step(i, carry):
        bvs, bis, curs = carry
        new_bvs, new_bis, new_curs = [], [], []
        for j in range(_U):
            v = buf[pl.ds((i * _U + j) * _L, _L)]
            take = v > bvs[j]
            new_bvs.append(jnp.maximum(bvs[j], v))
            new_bis.append(jnp.where(take, curs[j], bis[j]))
            new_curs.append(curs[j] + _U * _L)
        return tuple(new_bvs), tuple(new_bis), tuple(new_curs)

    init = (
        tuple(jnp.full((_L,), _NEG, jnp.float32) for _ in range(_U)),
        tuple(jnp.zeros((_L,), jnp.int32) for _ in range(_U)),
        tuple(base + j * _L + lanes for j in range(_U)),
    )
    bvs, bis, _ = lax.fori_loop(0, _CHUNK // (_U * _L), step, init)

    bvs, bis = list(bvs), list(bis)
    while len(bvs) > 1:
        nv, ni = [], []
        for k in range(0, len(bvs) - 1, 2):
            va, vb = bvs[k], bvs[k + 1]
            ia, ib = bis[k], bis[k + 1]
            take_b = (vb > va) | ((vb == va) & (ib < ia))
            nv.append(jnp.where(take_b, vb, va))
            ni.append(jnp.where(take_b, ib, ia))
        if len(bvs) % 2:
            nv.append(bvs[-1])
            ni.append(bis[-1])
        bvs, bis = nv, ni

    val_s[...] = bvs[0]
    idx_s[...] = bis[0]
    pltpu.sync_copy(val_s, vals_out.at[pl.ds(wid * _L, _L)])
    pltpu.sync_copy(idx_s, idxs_out.at[pl.ds(wid * _L, _L)])


@functools.lru_cache(maxsize=None)
def _build_sc_scan():
    return pl.kernel(
        _sc_scan_body,
        out_type=(
            jax.ShapeDtypeStruct((_NW * _L,), jnp.float32),
            jax.ShapeDtypeStruct((_NW * _L,), jnp.int32),
        ),
        mesh=plsc.VectorSubcoreMesh(
            core_axis_name="c", subcore_axis_name="s",
            num_cores=_NC, num_subcores=_NS,
        ),
        scratch_types=(
            pltpu.VMEM((_CHUNK,), jnp.float32),
            pltpu.VMEM((_L,), jnp.float32),
            pltpu.VMEM((_L,), jnp.int32),
        ),
    )


def _tc_finish_body(v_ref, i_ref, score_ref, label_ref):
    v = v_ref[...]
    idx = i_ref[...]
    m = jnp.max(v)
    sel = jnp.where(v == m, idx, jnp.int32(2**31 - 1))
    mi = jnp.min(sel)
    mi = (mi >> 7) * _C + (mi & (_CP - 1))
    score_ref[...] = 1.0 / (1.0 + jnp.exp(-jnp.max(v, keepdims=True).reshape(1, 1)))
    label_ref[...] = jnp.full((1, 1), mi % _C, jnp.int32)


def _tc_finish(vals, idxs):
    return pl.pallas_call(
        _tc_finish_body,
        out_shape=(
            jax.ShapeDtypeStruct((1, 1), jnp.float32),
            jax.ShapeDtypeStruct((1, 1), jnp.int32),
        ),
    )(vals, idxs)


def kernel(pred_logits, pred_boxes):
    del pred_boxes
    xp = _tc_pad(pred_logits).reshape(-1)
    vals, idxs = _build_sc_scan()(xp)
    score, label = _tc_finish(vals.reshape(4, 128), idxs.reshape(4, 128))
    return (score.reshape(1), label.reshape(1))


# speedup vs baseline: 2.0711x; 2.0711x over previous
import functools

import jax
import jax.numpy as jnp
from jax import lax
from jax.experimental import pallas as pl
from jax.experimental.pallas import tpu as pltpu
from jax.experimental.pallas import tpu_sc as plsc

_R = 20000
_C = 91
_CP = 128
_NP = _R * _CP
_NC = 2
_NS = 16
_NW = _NC * _NS
_L = 16
_U = 8
_CHUNK = _NP // _NW
_NEG = -1.0e30
_BR = 2000


def _sc_scan_body(x_hbm, vals_out, idxs_out, buf, val_s, idx_s):
    cid = lax.axis_index("c")
    sid = lax.axis_index("s")
    wid = sid * _NC + cid
    base = wid * _CHUNK
    pltpu.sync_copy(x_hbm.at[pl.ds(base, _CHUNK)], buf)

    lanes = lax.iota(jnp.int32, 16)

    def step(i, carry):
        bvs, bis, curs = carry
        new_bvs, new_bis, new_curs = [], [], []
        for j in range(_U):
            v = buf[pl.ds((i * _U + j) * _L, _L)]
            take = v > bvs[j]
            new_bvs.append(jnp.maximum(bvs[j], v))
            new_bis.append(jnp.where(take, curs[j], bis[j]))
            new_curs.append(curs[j] + _U * _L)
        return tuple(new_bvs), tuple(new_bis), tuple(new_curs)

    init = (
        tuple(jnp.full((_L,), _NEG, jnp.float32) for _ in range(_U)),
        tuple(jnp.zeros((_L,), jnp.int32) for _ in range(_U)),
        tuple(base + j * _L + lanes for j in range(_U)),
    )
    bvs, bis, _ = lax.fori_loop(0, _CHUNK // (_U * _L), step, init)

    bvs, bis = list(bvs), list(bis)
    while len(bvs) > 1:
        nv, ni = [], []
        for k in range(0, len(bvs) - 1, 2):
            va, vb = bvs[k], bvs[k + 1]
            ia, ib = bis[k], bis[k + 1]
            take_b = (vb > va) | ((vb == va) & (ib < ia))
            nv.append(jnp.where(take_b, vb, va))
            ni.append(jnp.where(take_b, ib, ia))
        if len(bvs) % 2:
            nv.append(bvs[-1])
            ni.append(bis[-1])
        bvs, bis = nv, ni

    val_s[...] = bvs[0]
    idx_s[...] = bis[0]
    pltpu.sync_copy(val_s, vals_out.at[pl.ds(wid * _L, _L)])
    pltpu.sync_copy(idx_s, idxs_out.at[pl.ds(wid * _L, _L)])


@functools.lru_cache(maxsize=None)
def _build_sc_scan():
    return pl.kernel(
        _sc_scan_body,
        out_type=(
            jax.ShapeDtypeStruct((_NW * _L,), jnp.float32),
            jax.ShapeDtypeStruct((_NW * _L,), jnp.int32),
        ),
        mesh=plsc.VectorSubcoreMesh(
            core_axis_name="c", subcore_axis_name="s",
            num_cores=_NC, num_subcores=_NS,
        ),
        scratch_types=(
            pltpu.VMEM((_CHUNK,), jnp.float32),
            pltpu.VMEM((_L,), jnp.float32),
            pltpu.VMEM((_L,), jnp.int32),
        ),
    )


def _tc_finish_body(v_ref, i_ref, score_ref, label_ref):
    v = v_ref[...]
    idx = i_ref[...]
    m = jnp.max(v)
    sel = jnp.where(v == m, idx, jnp.int32(2**31 - 1))
    mi = jnp.min(sel)
    mi = (mi >> 7) * _C + (mi & (_CP - 1))
    score_ref[...] = 1.0 / (1.0 + jnp.exp(-jnp.max(v, keepdims=True).reshape(1, 1)))
    label_ref[...] = jnp.full((1, 1), mi % _C, jnp.int32)


def _tc_finish(vals, idxs):
    return pl.pallas_call(
        _tc_finish_body,
        out_shape=(
            jax.ShapeDtypeStruct((1, 1), jnp.float32),
            jax.ShapeDtypeStruct((1, 1), jnp.int32),
        ),
    )(vals, idxs)


def kernel(pred_logits, pred_boxes):
    del pred_boxes
    xp = jnp.pad(
        pred_logits[0], ((0, 0), (0, _CP - _C)), constant_values=_NEG
    ).reshape(-1)
    vals, idxs = _build_sc_scan()(xp)
    score, label = _tc_finish(vals.reshape(4, 128), idxs.reshape(4, 128))
    return (score.reshape(1), label.reshape(1))


# speedup vs baseline: 2.4988x; 1.2065x over previous
import functools

import jax
import jax.numpy as jnp
from jax import lax
from jax.experimental import pallas as pl
from jax.experimental.pallas import tpu as pltpu
from jax.experimental.pallas import tpu_sc as plsc

_R = 20000
_C = 91
_NC = 2
_NS = 16
_NW = _NC * _NS
_L = 16
_KC = 3
_PH = 2
_NU = _KC * _PH


def _sc_scan_body(y_hbm, vals_out, idxs_out, buf0, buf1, buf2, val_s, idx_s):
    cid = lax.axis_index("c")
    sid = lax.axis_index("s")
    wid = sid * _NC + cid

    bufs = (buf0, buf1, buf2)
    classes = [
        wid,
        wid + _NW,
        jnp.minimum(wid + 2 * _NW, _C - 1),
    ]
    for k, c in enumerate(classes):
        pltpu.sync_copy(y_hbm.at[pl.ds(c * _R, _R)], bufs[k])

    lanes91 = lax.iota(jnp.int32, 16) * _C

    def step(i, carry):
        bvs, bis, curs = carry
        new_bvs, new_bis, new_curs = [], [], []
        for j in range(_NU):
            k, q = divmod(j, _PH)
            v = bufs[k][pl.ds((i * _PH + q) * _L, _L)]
            take = v > bvs[j]
            new_bvs.append(jnp.maximum(bvs[j], v))
            new_bis.append(jnp.where(take, curs[j], bis[j]))
            new_curs.append(curs[j] + _PH * _L * _C)
        return tuple(new_bvs), tuple(new_bis), tuple(new_curs)

    init = (
        tuple(jnp.full((_L,), -1.0e30, jnp.float32) for _ in range(_NU)),
        tuple(jnp.zeros((_L,), jnp.int32) for _ in range(_NU)),
        tuple(
            classes[j // _PH] + (j % _PH) * _L * _C + lanes91
            for j in range(_NU)
        ),
    )
    bvs, bis, _ = lax.fori_loop(0, _R // (_PH * _L), step, init)

    bvs, bis = list(bvs), list(bis)
    while len(bvs) > 1:
        nv, ni = [], []
        for k in range(0, len(bvs) - 1, 2):
            va, vb = bvs[k], bvs[k + 1]
            ia, ib = bis[k], bis[k + 1]
            take_b = (vb > va) | ((vb == va) & (ib < ia))
            nv.append(jnp.where(take_b, vb, va))
            ni.append(jnp.where(take_b, ib, ia))
        if len(bvs) % 2:
            nv.append(bvs[-1])
            ni.append(bis[-1])
        bvs, bis = nv, ni

    val_s[...] = bvs[0]
    idx_s[...] = bis[0]
    pltpu.sync_copy(val_s, vals_out.at[pl.ds(wid * _L, _L)])
    pltpu.sync_copy(idx_s, idxs_out.at[pl.ds(wid * _L, _L)])


@functools.lru_cache(maxsize=None)
def _build_sc_scan():
    return pl.kernel(
        _sc_scan_body,
        out_type=(
            jax.ShapeDtypeStruct((_NW * _L,), jnp.float32),
            jax.ShapeDtypeStruct((_NW * _L,), jnp.int32),
        ),
        mesh=plsc.VectorSubcoreMesh(
            core_axis_name="c", subcore_axis_name="s",
            num_cores=_NC, num_subcores=_NS,
        ),
        scratch_types=(
            pltpu.VMEM((_R,), jnp.float32),
            pltpu.VMEM((_R,), jnp.float32),
            pltpu.VMEM((_R,), jnp.float32),
            pltpu.VMEM((_L,), jnp.float32),
            pltpu.VMEM((_L,), jnp.int32),
        ),
    )


def _tc_finish_body(v_ref, i_ref, score_ref, label_ref):
    v = v_ref[...]
    idx = i_ref[...]
    m = jnp.max(v)
    sel = jnp.where(v == m, idx, jnp.int32(2**31 - 1))
    mi = jnp.min(sel, keepdims=True).reshape(1, 1)
    score_ref[...] = 1.0 / (1.0 + jnp.exp(-jnp.max(v, keepdims=True).reshape(1, 1)))
    label_ref[...] = mi % _C


def _tc_finish(vals, idxs):
    return pl.pallas_call(
        _tc_finish_body,
        out_shape=(
            jax.ShapeDtypeStruct((1, 1), jnp.float32),
            jax.ShapeDtypeStruct((1, 1), jnp.int32),
        ),
    )(vals, idxs)


def kernel(pred_logits, pred_boxes):
    del pred_boxes
    yf = pred_logits[0].T.reshape(-1)
    vals, idxs = _build_sc_scan()(yf)
    score, label = _tc_finish(vals.reshape(4, 128), idxs.reshape(4, 128))
    return (score.reshape(1), label.reshape(1))


# speedup vs baseline: 2.8046x; 1.1224x over previous
import functools

import jax
import jax.numpy as jnp
from jax import lax
from jax.experimental import pallas as pl
from jax.experimental.pallas import tpu as pltpu
from jax.experimental.pallas import tpu_sc as plsc

_R = 20000
_C = 91
_NC = 2
_NS = 16
_NW = _NC * _NS
_L = 16
_KC = 3
_PH = 2
_NU = _KC * _PH


def _sc_scan_body(y_hbm, vals_out, idxs_out, buf0, buf1, buf2, val_s, idx_s):
    cid = lax.axis_index("c")
    sid = lax.axis_index("s")
    wid = sid * _NC + cid

    bufs = (buf0, buf1, buf2)
    classes = [
        wid,
        wid + _NW,
        jnp.minimum(wid + 2 * _NW, _C - 1),
    ]
    for k, c in enumerate(classes):
        pltpu.sync_copy(y_hbm.at[pl.ds(c, 1), :], bufs[k])

    lanes91 = lax.iota(jnp.int32, 16) * _C

    def step(i, carry):
        bvs, bis, curs = carry
        new_bvs, new_bis, new_curs = [], [], []
        for j in range(_NU):
            k, q = divmod(j, _PH)
            v = bufs[k][0, pl.ds((i * _PH + q) * _L, _L)]
            take = v > bvs[j]
            new_bvs.append(jnp.maximum(bvs[j], v))
            new_bis.append(jnp.where(take, curs[j], bis[j]))
            new_curs.append(curs[j] + _PH * _L * _C)
        return tuple(new_bvs), tuple(new_bis), tuple(new_curs)

    init = (
        tuple(jnp.full((_L,), -1.0e30, jnp.float32) for _ in range(_NU)),
        tuple(jnp.zeros((_L,), jnp.int32) for _ in range(_NU)),
        tuple(
            classes[j // _PH] + (j % _PH) * _L * _C + lanes91
            for j in range(_NU)
        ),
    )
    bvs, bis, _ = lax.fori_loop(0, _R // (_PH * _L), step, init)

    bvs, bis = list(bvs), list(bis)
    while len(bvs) > 1:
        nv, ni = [], []
        for k in range(0, len(bvs) - 1, 2):
            va, vb = bvs[k], bvs[k + 1]
            ia, ib = bis[k], bis[k + 1]
            take_b = (vb > va) | ((vb == va) & (ib < ia))
            nv.append(jnp.where(take_b, vb, va))
            ni.append(jnp.where(take_b, ib, ia))
        if len(bvs) % 2:
            nv.append(bvs[-1])
            ni.append(bis[-1])
        bvs, bis = nv, ni

    val_s[...] = bvs[0]
    idx_s[...] = bis[0]
    pltpu.sync_copy(val_s, vals_out.at[pl.ds(wid * _L, _L)])
    pltpu.sync_copy(idx_s, idxs_out.at[pl.ds(wid * _L, _L)])


@functools.lru_cache(maxsize=None)
def _build_sc_scan():
    return pl.kernel(
        _sc_scan_body,
        out_type=(
            jax.ShapeDtypeStruct((_NW * _L,), jnp.float32),
            jax.ShapeDtypeStruct((_NW * _L,), jnp.int32),
        ),
        name="sc_argmax_scan",
        mesh=plsc.VectorSubcoreMesh(
            core_axis_name="c", subcore_axis_name="s",
            num_cores=_NC, num_subcores=_NS,
        ),
        scratch_types=(
            pltpu.VMEM((1, _R), jnp.float32),
            pltpu.VMEM((1, _R), jnp.float32),
            pltpu.VMEM((1, _R), jnp.float32),
            pltpu.VMEM((_L,), jnp.float32),
            pltpu.VMEM((_L,), jnp.int32),
        ),
    )


def _tc_finish_body(v_ref, i_ref, score_ref, label_ref):
    v = v_ref[...]
    idx = i_ref[...]
    m = jnp.max(v)
    sel = jnp.where(v == m, idx, jnp.int32(2**31 - 1))
    mi = jnp.min(sel, keepdims=True).reshape(1, 1)
    score_ref[...] = 1.0 / (1.0 + jnp.exp(-jnp.max(v, keepdims=True).reshape(1, 1)))
    label_ref[...] = mi % _C


def _tc_finish(vals, idxs):
    return pl.pallas_call(
        _tc_finish_body,
        out_shape=(
            jax.ShapeDtypeStruct((1, 1), jnp.float32),
            jax.ShapeDtypeStruct((1, 1), jnp.int32),
        ),
    )(vals, idxs)


def kernel(pred_logits, pred_boxes):
    del pred_boxes
    y2d = pred_logits[0].T
    vals, idxs = _build_sc_scan()(y2d)
    score, label = _tc_finish(vals.reshape(4, 128), idxs.reshape(4, 128))
    return (score.reshape(1), label.reshape(1))


# speedup vs baseline: 5.9077x; 2.1064x over previous
import functools

import jax
import jax.numpy as jnp
from jax import lax
from jax.experimental import pallas as pl
from jax.experimental.pallas import tpu as pltpu
from jax.experimental.pallas import tpu_sc as plsc

_R = 20000
_C = 91
_NC = 2
_NS = 16
_NW = _NC * _NS
_L = 16
_KC = 3
_PH = 2
_NU = _KC * _PH


def _sc_scan_body(y_hbm, vals_out, idxs_out, buf0, buf1, buf2, val_s, idx_s):
    cid = lax.axis_index("c")
    sid = lax.axis_index("s")
    wid = sid * _NC + cid

    bufs = (buf0, buf1, buf2)
    classes = [
        wid,
        wid + _NW,
        jnp.minimum(wid + 2 * _NW, _C - 1),
    ]
    for k, c in enumerate(classes):
        pltpu.sync_copy(y_hbm.at[pl.ds(c, 1), pl.ds(0, 1), :], bufs[k])

    lanes91 = lax.iota(jnp.int32, 16) * _C

    def step(i, carry):
        bvs, bis, curs = carry
        new_bvs, new_bis, new_curs = [], [], []
        for j in range(_NU):
            k, q = divmod(j, _PH)
            v = bufs[k][0, 0, pl.ds((i * _PH + q) * _L, _L)]
            take = v > bvs[j]
            new_bvs.append(jnp.maximum(bvs[j], v))
            new_bis.append(jnp.where(take, curs[j], bis[j]))
            new_curs.append(curs[j] + _PH * _L * _C)
        return tuple(new_bvs), tuple(new_bis), tuple(new_curs)

    init = (
        tuple(jnp.full((_L,), -1.0e30, jnp.float32) for _ in range(_NU)),
        tuple(jnp.zeros((_L,), jnp.int32) for _ in range(_NU)),
        tuple(
            classes[j // _PH] + (j % _PH) * _L * _C + lanes91
            for j in range(_NU)
        ),
    )
    bvs, bis, _ = lax.fori_loop(0, _R // (_PH * _L), step, init)

    bvs, bis = list(bvs), list(bis)
    while len(bvs) > 1:
        nv, ni = [], []
        for k in range(0, len(bvs) - 1, 2):
            va, vb = bvs[k], bvs[k + 1]
            ia, ib = bis[k], bis[k + 1]
            take_b = (vb > va) | ((vb == va) & (ib < ia))
            nv.append(jnp.where(take_b, vb, va))
            ni.append(jnp.where(take_b, ib, ia))
        if len(bvs) % 2:
            nv.append(bvs[-1])
            ni.append(bis[-1])
        bvs, bis = nv, ni

    val_s[...] = bvs[0]
    idx_s[...] = bis[0]
    pltpu.sync_copy(val_s, vals_out.at[pl.ds(wid * _L, _L)])
    pltpu.sync_copy(idx_s, idxs_out.at[pl.ds(wid * _L, _L)])


@functools.lru_cache(maxsize=None)
def _build_sc_scan():
    return pl.kernel(
        _sc_scan_body,
        out_type=(
            jax.ShapeDtypeStruct((_NW * _L,), jnp.float32),
            jax.ShapeDtypeStruct((_NW * _L,), jnp.int32),
        ),
        name="sc_argmax_scan",
        mesh=plsc.VectorSubcoreMesh(
            core_axis_name="c", subcore_axis_name="s",
            num_cores=_NC, num_subcores=_NS,
        ),
        scratch_types=(
            pltpu.VMEM((1, 1, _R), jnp.float32),
            pltpu.VMEM((1, 1, _R), jnp.float32),
            pltpu.VMEM((1, 1, _R), jnp.float32),
            pltpu.VMEM((_L,), jnp.float32),
            pltpu.VMEM((_L,), jnp.int32),
        ),
    )


def _tc_finish_body(v_ref, i_ref, score_ref, label_ref):
    v = v_ref[...]
    idx = i_ref[...]
    m = jnp.max(v)
    sel = jnp.where(v == m, idx, jnp.int32(2**31 - 1))
    mi = jnp.min(sel, keepdims=True).reshape(1, 1)
    score_ref[...] = 1.0 / (1.0 + jnp.exp(-jnp.max(v, keepdims=True).reshape(1, 1)))
    label_ref[...] = mi % _C


def _tc_finish(vals, idxs):
    return pl.pallas_call(
        _tc_finish_body,
        out_shape=(
            jax.ShapeDtypeStruct((1, 1), jnp.float32),
            jax.ShapeDtypeStruct((1, 1), jnp.int32),
        ),
    )(vals, idxs)


def kernel(pred_logits, pred_boxes):
    del pred_boxes
    yt = jnp.transpose(pred_logits, (2, 0, 1))
    vals, idxs = _build_sc_scan()(yt)
    score, label = _tc_finish(vals.reshape(4, 128), idxs.reshape(4, 128))
    return (score.reshape(1), label.reshape(1))
